# baseline (speedup 1.0000x reference)
import functools

import jax
import jax.numpy as jnp
import numpy as np
from jax import lax
from jax.experimental import pallas as pl
from jax.experimental.pallas import tpu as pltpu
from jax.experimental.pallas import tpu_sc as plsc


def _table_factors(t, dim, split):
    i = np.arange(dim, dtype=np.float64)
    w = np.power(10000.0, -(i - (i % 2)) / dim)
    ph = (i % 2) * (np.pi / 2)
    h = np.arange(t // split, dtype=np.float64)[:, None]
    l = np.arange(split, dtype=np.float64)[:, None]
    hs = np.sin(h * split * w)
    hc = np.cos(h * split * w)
    ls = np.sin(l * w + ph)
    lc = np.cos(l * w + ph)
    return tuple(jnp.asarray(x, dtype=jnp.float32) for x in (hs, hc, ls, lc))


def _sc_broadcast_rows(table, nb, t, dim):
    info = plsc.get_sparse_core_info()
    nw = info.num_cores * info.num_subcores
    rows_per_w = t // nw
    chunk = min(rows_per_w, 32)
    n_chunks = rows_per_w // chunk
    mesh = plsc.VectorSubcoreMesh(core_axis_name="c", subcore_axis_name="s")

    @functools.partial(
        pl.kernel,
        mesh=mesh,
        out_type=jax.ShapeDtypeStruct((nb * t, dim), jnp.float32),
        scratch_types=[
            pltpu.VMEM((2, chunk, dim), jnp.float32),
            pltpu.SemaphoreType.DMA,
            pltpu.SemaphoreType.DMA,
            pltpu.SemaphoreType.DMA,
        ],
    )
    def k(table_hbm, out_hbm, buf, ld_sem, st_sem0, st_sem1):
        wid = lax.axis_index("s") * info.num_cores + lax.axis_index("c")
        base = wid * rows_per_w
        st_sems = (st_sem0, st_sem1)

        def start_load(c):
            return pltpu.async_copy(
                table_hbm.at[pl.ds(base + c * chunk, chunk)], buf.at[c % 2], ld_sem
            )

        loads = [None] * n_chunks
        stores = [[] for _ in range(n_chunks)]
        loads[0] = start_load(0)
        for c in range(n_chunks):
            loads[c].wait()
            if c + 1 < n_chunks:
                if c >= 1:
                    for d in stores[c - 1]:
                        d.wait()
                loads[c + 1] = start_load(c + 1)
            row0 = base + c * chunk
            for bb in range(nb):
                stores[c].append(
                    pltpu.async_copy(
                        buf.at[c % 2],
                        out_hbm.at[pl.ds(bb * t + row0, chunk)],
                        st_sems[c % 2],
                    )
                )
        for c in (n_chunks - 2, n_chunks - 1):
            if c >= 0:
                for d in stores[c]:
                    d.wait()

    return k(table).reshape(nb, t, dim)


def _tc_tablegen(hs, hc, ls, lc, t, dim, split):
    def body(hs_ref, hc_ref, ls_ref, lc_ref, out_ref):
        out_ref[...] = hs_ref[0] * lc_ref[...] + hc_ref[0] * ls_ref[...]

    n_h = t // split
    return pl.pallas_call(
        body,
        grid=(n_h,),
        in_specs=[
            pl.BlockSpec((1, 1, dim), lambda i: (i, 0, 0)),
            pl.BlockSpec((1, 1, dim), lambda i: (i, 0, 0)),
            pl.BlockSpec((split, dim), lambda i: (0, 0)),
            pl.BlockSpec((split, dim), lambda i: (0, 0)),
        ],
        out_specs=pl.BlockSpec((split, dim), lambda i: (i, 0)),
        out_shape=jax.ShapeDtypeStruct((t, dim), jnp.float32),
    )(hs[:, None, :], hc[:, None, :], ls, lc)


def kernel(inputs):
    b, t, dim = inputs.shape
    split = 64
    hs, hc, ls, lc = _table_factors(t, dim, split)
    table = _tc_tablegen(hs, hc, ls, lc, t, dim, split)
    return _sc_broadcast_rows(table, b, t, dim)


# speedup vs baseline: 1.4732x; 1.4732x over previous
import functools

import jax
import jax.numpy as jnp
import numpy as np
from jax import lax
from jax.experimental import pallas as pl
from jax.experimental.pallas import tpu as pltpu
from jax.experimental.pallas import tpu_sc as plsc


def _table_factors(t, dim, split):
    i = np.arange(dim, dtype=np.float64)
    w = np.power(10000.0, -(i - (i % 2)) / dim)
    ph = (i % 2) * (np.pi / 2)
    h = np.arange(t // split, dtype=np.float64)[:, None]
    l = np.arange(split, dtype=np.float64)[:, None]
    hs = np.sin(h * split * w)
    hc = np.cos(h * split * w)
    ls = np.sin(l * w + ph)
    lc = np.cos(l * w + ph)
    return tuple(jnp.asarray(x, dtype=jnp.float32) for x in (hs, hc, ls, lc))


def _sc_broadcast_rows(table, nb, t, dim):
    info = plsc.get_sparse_core_info()
    nw = info.num_cores * info.num_subcores
    rows_per_w = t // nw
    chunk = min(rows_per_w, 32)
    n_chunks = rows_per_w // chunk
    mesh = plsc.VectorSubcoreMesh(core_axis_name="c", subcore_axis_name="s")

    @functools.partial(
        pl.kernel,
        mesh=mesh,
        out_type=jax.ShapeDtypeStruct((nb * t, dim), jnp.float32),
        scratch_types=[
            pltpu.VMEM((2, chunk, dim), jnp.float32),
            pltpu.SemaphoreType.DMA,
            pltpu.SemaphoreType.DMA,
            pltpu.SemaphoreType.DMA,
        ],
    )
    def k(table_hbm, out_hbm, buf, ld_sem, st_sem0, st_sem1):
        wid = lax.axis_index("s") * info.num_cores + lax.axis_index("c")
        base = wid * rows_per_w
        st_sems = (st_sem0, st_sem1)

        def start_load(c):
            return pltpu.async_copy(
                table_hbm.at[pl.ds(base + c * chunk, chunk)], buf.at[c % 2], ld_sem
            )

        loads = [None] * n_chunks
        stores = [[] for _ in range(n_chunks)]
        loads[0] = start_load(0)
        for c in range(n_chunks):
            loads[c].wait()
            if c + 1 < n_chunks:
                if c >= 1:
                    for d in stores[c - 1]:
                        d.wait()
                loads[c + 1] = start_load(c + 1)
            row0 = base + c * chunk
            for bb in range(nb):
                stores[c].append(
                    pltpu.async_copy(
                        buf.at[c % 2],
                        out_hbm.at[pl.ds(bb * t + row0, chunk)],
                        st_sems[c % 2],
                    )
                )
        for c in (n_chunks - 2, n_chunks - 1):
            if c >= 0:
                for d in stores[c]:
                    d.wait()

    return k(table).reshape(nb, t, dim)


def _tc_tablegen(hs, hc, ls, lc, t, dim, split):
    h_per_step = 8

    def body(hs_ref, hc_ref, ls_ref, lc_ref, out_ref):
        for k in range(h_per_step):
            out_ref[k * split:(k + 1) * split] = (
                hs_ref[k] * lc_ref[...] + hc_ref[k] * ls_ref[...]
            )

    n_h = t // split
    return pl.pallas_call(
        body,
        grid=(n_h // h_per_step,),
        in_specs=[
            pl.BlockSpec((h_per_step, 1, dim), lambda i: (i, 0, 0)),
            pl.BlockSpec((h_per_step, 1, dim), lambda i: (i, 0, 0)),
            pl.BlockSpec((split, dim), lambda i: (0, 0)),
            pl.BlockSpec((split, dim), lambda i: (0, 0)),
        ],
        out_specs=pl.BlockSpec((h_per_step * split, dim), lambda i: (i, 0)),
        out_shape=jax.ShapeDtypeStruct((t, dim), jnp.float32),
    )(hs[:, None, :], hc[:, None, :], ls, lc)


def kernel(inputs):
    b, t, dim = inputs.shape
    split = 64
    hs, hc, ls, lc = _table_factors(t, dim, split)
    table = _tc_tablegen(hs, hc, ls, lc, t, dim, split)
    return _sc_broadcast_rows(table, b, t, dim)


# speedup vs baseline: 1.5058x; 1.0221x over previous
import functools

import jax
import jax.numpy as jnp
import numpy as np
from jax import lax
from jax.experimental import pallas as pl
from jax.experimental.pallas import tpu as pltpu
from jax.experimental.pallas import tpu_sc as plsc


def _table_factors(t, dim, split):
    i = np.arange(dim, dtype=np.float64)
    w = np.power(10000.0, -(i - (i % 2)) / dim)
    ph = (i % 2) * (np.pi / 2)
    h = np.arange(t // split, dtype=np.float64)[:, None]
    l = np.arange(split, dtype=np.float64)[:, None]
    hs = np.sin(h * split * w)
    hc = np.cos(h * split * w)
    ls = np.sin(l * w + ph)
    lc = np.cos(l * w + ph)
    return tuple(jnp.asarray(x, dtype=jnp.float32) for x in (hs, hc, ls, lc))


def _sc_broadcast_rows(table, nb, t, dim):
    info = plsc.get_sparse_core_info()
    nw = info.num_cores * info.num_subcores
    rows_per_w = t // nw
    chunk = min(rows_per_w, 32)
    n_chunks = rows_per_w // chunk
    mesh = plsc.VectorSubcoreMesh(core_axis_name="c", subcore_axis_name="s")

    @functools.partial(
        pl.kernel,
        mesh=mesh,
        out_type=jax.ShapeDtypeStruct((nb * t, dim), jnp.float32),
        scratch_types=[
            pltpu.VMEM((3, chunk, dim), jnp.float32),
            pltpu.SemaphoreType.DMA,
            pltpu.SemaphoreType.DMA,
            pltpu.SemaphoreType.DMA,
            pltpu.SemaphoreType.DMA,
        ],
    )
    def k(table_hbm, out_hbm, buf, ld_sem, st_sem0, st_sem1, st_sem2):
        wid = lax.axis_index("s") * info.num_cores + lax.axis_index("c")
        base = wid * rows_per_w
        st_sems = (st_sem0, st_sem1, st_sem2)
        nbuf = 3

        def start_load(c):
            return pltpu.async_copy(
                table_hbm.at[pl.ds(base + c * chunk, chunk)], buf.at[c % nbuf], ld_sem
            )

        loads = [None] * n_chunks
        stores = [[] for _ in range(n_chunks)]
        for c in range(min(nbuf - 1, n_chunks)):
            loads[c] = start_load(c)
        for c in range(n_chunks):
            loads[c].wait()
            nxt = c + nbuf - 1
            if nxt < n_chunks:
                if c >= 1:
                    for d in stores[c - 1]:
                        d.wait()
                    stores[c - 1] = []
                loads[nxt] = start_load(nxt)
            row0 = base + c * chunk
            for bb in range(nb):
                stores[c].append(
                    pltpu.async_copy(
                        buf.at[c % nbuf],
                        out_hbm.at[pl.ds(bb * t + row0, chunk)],
                        st_sems[c % nbuf],
                    )
                )
        for c in range(n_chunks):
            for d in stores[c]:
                d.wait()

    return k(table).reshape(nb, t, dim)


def _tc_tablegen(hs, hc, ls, lc, t, dim, split):
    h_per_step = 8

    def body(hs_ref, hc_ref, ls_ref, lc_ref, out_ref):
        for k in range(h_per_step):
            out_ref[k * split:(k + 1) * split] = (
                hs_ref[k] * lc_ref[...] + hc_ref[k] * ls_ref[...]
            )

    n_h = t // split
    return pl.pallas_call(
        body,
        grid=(n_h // h_per_step,),
        in_specs=[
            pl.BlockSpec((h_per_step, 1, dim), lambda i: (i, 0, 0)),
            pl.BlockSpec((h_per_step, 1, dim), lambda i: (i, 0, 0)),
            pl.BlockSpec((split, dim), lambda i: (0, 0)),
            pl.BlockSpec((split, dim), lambda i: (0, 0)),
        ],
        out_specs=pl.BlockSpec((h_per_step * split, dim), lambda i: (i, 0)),
        out_shape=jax.ShapeDtypeStruct((t, dim), jnp.float32),
    )(hs[:, None, :], hc[:, None, :], ls, lc)


def kernel(inputs):
    b, t, dim = inputs.shape
    split = 64
    hs, hc, ls, lc = _table_factors(t, dim, split)
    table = _tc_tablegen(hs, hc, ls, lc, t, dim, split)
    return _sc_broadcast_rows(table, b, t, dim)
